# baseline (speedup 1.0000x reference)
import functools

import jax
import jax.numpy as jnp
from jax import lax
from jax.experimental import pallas as pl
from jax.experimental.pallas import tpu as pltpu
from jax.experimental.pallas import tpu_sc as plsc

V = 100000
P = 8192
H = 1024
B = 4
S = 2048

NC = 2
NS = 16
NW = NC * NS
NT = B * S
NSPLIT = 2
NTH = NT // NSPLIT
BH = B // NSPLIT
TPW = NTH // NW
CH = 16
NBUF = 4
LOOKAHEAD = 2
NCHUNK = TPW // CH

TOK_BLK = 2048
POS_BLKS = S // TOK_BLK


def _gather_body(ids_hbm, word_hbm, out_hbm,
                 idx_v, rows_0, rows_1, rows_2, rows_3,
                 gsem_0, gsem_1, gsem_2, gsem_3,
                 osem_0, osem_1, osem_2, osem_3):
    cid = lax.axis_index("c")
    sid = lax.axis_index("s")
    wid = sid * NC + cid
    base = pl.multiple_of(wid * TPW, TPW)

    pltpu.sync_copy(ids_hbm.at[pl.ds(base, TPW)], idx_v)

    bufs = [rows_0, rows_1, rows_2, rows_3]
    gsems = [gsem_0, gsem_1, gsem_2, gsem_3]
    osems = [osem_0, osem_1, osem_2, osem_3]

    def start_gather(c):
        b = c % NBUF
        return pltpu.async_copy(
            word_hbm.at[idx_v.at[pl.ds(c * CH, CH)]], bufs[b], gsems[b])

    ghandles = {c: start_gather(c) for c in range(min(LOOKAHEAD, NCHUNK))}
    ohandles = {}
    for c in range(NCHUNK):
        b = c % NBUF
        n = c + LOOKAHEAD
        if n < NCHUNK:
            prev = n - NBUF
            if prev >= 0:
                ohandles.pop(prev).wait()
            ghandles[n] = start_gather(n)
        ghandles.pop(c).wait()
        ohandles[c] = pltpu.async_copy(
            bufs[b], out_hbm.at[pl.ds(base + c * CH, CH)], osems[b])
    for h in ohandles.values():
        h.wait()


def _ln_body(g_ref, p_ref, tt_ref, gamma_ref, beta_ref, o_ref):
    x = g_ref[...] + p_ref[...] + tt_ref[...]
    mean = jnp.mean(x, axis=-1, keepdims=True)
    d = x - mean
    var = jnp.mean(d * d, axis=-1, keepdims=True)
    y = d * lax.rsqrt(var + jnp.float32(1e-5))
    o_ref[...] = y * gamma_ref[...] + beta_ref[...]


def _sc_gather(ids_half, word):
    mesh = plsc.VectorSubcoreMesh(core_axis_name="c", subcore_axis_name="s")
    return functools.partial(
        pl.kernel,
        out_type=jax.ShapeDtypeStruct((NTH, H), jnp.float32),
        mesh=mesh,
        scratch_types=[
            pltpu.VMEM((TPW,), jnp.int32)]
            + [pltpu.VMEM((CH, H), jnp.float32) for _ in range(NBUF)]
            + [pltpu.SemaphoreType.DMA for _ in range(2 * NBUF)],
        compiler_params=pltpu.CompilerParams(needs_layout_passes=False),
    )(_gather_body)(ids_half, word)


def _tc_ln(gathered, pos, tt_row, gamma2, beta2):
    return pl.pallas_call(
        _ln_body,
        grid=(POS_BLKS, BH),
        in_specs=[
            pl.BlockSpec((TOK_BLK, H), lambda p, b: (b * POS_BLKS + p, 0)),
            pl.BlockSpec((TOK_BLK, H), lambda p, b: (p, 0)),
            pl.BlockSpec((1, H), lambda p, b: (0, 0)),
            pl.BlockSpec((1, H), lambda p, b: (0, 0)),
            pl.BlockSpec((1, H), lambda p, b: (0, 0)),
        ],
        out_specs=pl.BlockSpec((TOK_BLK, H), lambda p, b: (b * POS_BLKS + p, 0)),
        out_shape=jax.ShapeDtypeStruct((NTH, H), jnp.float32),
    )(gathered, pos, tt_row, gamma2, beta2)


@jax.jit
def _run(ids_flat, word, pos, tt, gamma, beta):
    tt_row = tt[0:1, :]
    gamma2 = gamma.reshape(1, H)
    beta2 = beta.reshape(1, H)
    pos_s = pos[:S]
    halves = [_sc_gather(ids_flat[h * NTH:(h + 1) * NTH], word)
              for h in range(NSPLIT)]
    outs = [_tc_ln(g, pos_s, tt_row, gamma2, beta2) for g in halves]
    return jnp.concatenate(outs, axis=0)


def kernel(input_ids, word_embeddings, position_embeddings,
           token_type_embeddings, ln_gamma, ln_beta):
    ids_flat = input_ids.reshape(NT).astype(jnp.int32)
    out = _run(ids_flat, word_embeddings, position_embeddings,
               token_type_embeddings, ln_gamma, ln_beta)
    return out.reshape(B, S, H)


# speedup vs baseline: 1.3440x; 1.3440x over previous
import functools

import jax
import jax.numpy as jnp
from jax import lax
from jax.experimental import pallas as pl
from jax.experimental.pallas import tpu as pltpu
from jax.experimental.pallas import tpu_sc as plsc

V = 100000
P = 8192
H = 1024
B = 4
S = 2048

NC = 2
NS = 16
NW = NC * NS
NT = B * S
NSPLIT = 1
NTH = NT // NSPLIT
BH = B // NSPLIT
TPW = NTH // NW
CH = 16
NBUF = 6
LOOKAHEAD = 3
NCHUNK = TPW // CH

TOK_BLK = 2048
POS_BLKS = S // TOK_BLK


def _gather_body(ids_hbm, word_hbm, out_hbm, idx_v, *bufs_and_sems):
    cid = lax.axis_index("c")
    sid = lax.axis_index("s")
    wid = sid * NC + cid
    base = pl.multiple_of(wid * TPW, TPW)

    pltpu.sync_copy(ids_hbm.at[pl.ds(base, TPW)], idx_v)

    bufs = list(bufs_and_sems[:NBUF])
    gsems = list(bufs_and_sems[NBUF:2 * NBUF])
    osems = list(bufs_and_sems[2 * NBUF:])

    def start_gather(c):
        b = c % NBUF
        return pltpu.async_copy(
            word_hbm.at[idx_v.at[pl.ds(c * CH, CH)]], bufs[b], gsems[b])

    ghandles = {c: start_gather(c) for c in range(min(LOOKAHEAD, NCHUNK))}
    ohandles = {}
    for c in range(NCHUNK):
        b = c % NBUF
        n = c + LOOKAHEAD
        if n < NCHUNK:
            prev = n - NBUF
            if prev >= 0:
                ohandles.pop(prev).wait()
            ghandles[n] = start_gather(n)
        ghandles.pop(c).wait()
        ohandles[c] = pltpu.async_copy(
            bufs[b], out_hbm.at[pl.ds(base + c * CH, CH)], osems[b])
    for h in ohandles.values():
        h.wait()


def _ln_body(g_ref, p_ref, tt_ref, gamma_ref, beta_ref, o_ref):
    x = g_ref[...] + p_ref[...] + tt_ref[...]
    mean = jnp.mean(x, axis=-1, keepdims=True)
    d = x - mean
    var = jnp.mean(d * d, axis=-1, keepdims=True)
    y = d * lax.rsqrt(var + jnp.float32(1e-5))
    o_ref[...] = y * gamma_ref[...] + beta_ref[...]


def _sc_gather(ids_half, word):
    mesh = plsc.VectorSubcoreMesh(core_axis_name="c", subcore_axis_name="s")
    return functools.partial(
        pl.kernel,
        out_type=jax.ShapeDtypeStruct((NTH, H), jnp.float32),
        mesh=mesh,
        scratch_types=[
            pltpu.VMEM((TPW,), jnp.int32)]
            + [pltpu.VMEM((CH, H), jnp.float32) for _ in range(NBUF)]
            + [pltpu.SemaphoreType.DMA for _ in range(2 * NBUF)],
        compiler_params=pltpu.CompilerParams(needs_layout_passes=False),
    )(_gather_body)(ids_half, word)


def _tc_ln(gathered, pos, tt_row, gamma2, beta2):
    return pl.pallas_call(
        _ln_body,
        grid=(POS_BLKS, BH),
        in_specs=[
            pl.BlockSpec((TOK_BLK, H), lambda p, b: (b * POS_BLKS + p, 0)),
            pl.BlockSpec((TOK_BLK, H), lambda p, b: (p, 0)),
            pl.BlockSpec((1, H), lambda p, b: (0, 0)),
            pl.BlockSpec((1, H), lambda p, b: (0, 0)),
            pl.BlockSpec((1, H), lambda p, b: (0, 0)),
        ],
        out_specs=pl.BlockSpec((TOK_BLK, H), lambda p, b: (b * POS_BLKS + p, 0)),
        out_shape=jax.ShapeDtypeStruct((NTH, H), jnp.float32),
    )(gathered, pos, tt_row, gamma2, beta2)


@jax.jit
def _run(ids_flat, word, pos, tt, gamma, beta):
    tt_row = tt[0:1, :]
    gamma2 = gamma.reshape(1, H)
    beta2 = beta.reshape(1, H)
    pos_s = pos[:S]
    halves = [_sc_gather(ids_flat[h * NTH:(h + 1) * NTH], word)
              for h in range(NSPLIT)]
    outs = [_tc_ln(g, pos_s, tt_row, gamma2, beta2) for g in halves]
    return jnp.concatenate(outs, axis=0)


def kernel(input_ids, word_embeddings, position_embeddings,
           token_type_embeddings, ln_gamma, ln_beta):
    ids_flat = input_ids.reshape(NT).astype(jnp.int32)
    out = _run(ids_flat, word_embeddings, position_embeddings,
               token_type_embeddings, ln_gamma, ln_beta)
    return out.reshape(B, S, H)
